# baseline (speedup 1.0000x reference)
import functools

import jax
import jax.numpy as jnp
from jax import lax
from jax.experimental import pallas as pl
from jax.experimental.pallas import tpu as pltpu
from jax.experimental.pallas import tpu_sc as plsc

B, S, D = 4, 2048, 768
VOCAB = 100000
PROMPT_LEN = 50
PID = 1

NW = 32
TOK_PER_W = (B * S) // NW
CHUNKS_PER_ROW = S // TOK_PER_W
SUB = 16
N_SUB = TOK_PER_W // SUB
NBUF = 8
PRIME = 4
MAX_P = 64
N_GRP = MAX_P // 16


def _worker_body(ids_hbm, table_hbm, prompt_hbm, out_hbm,
                 ids_v, idxc_v, rows_v, pos_v, pv_v, sem_g, sem_s, sem_p):
    wid = lax.axis_index("s") * 2 + lax.axis_index("c")
    b = wid // CHUNKS_PER_ROW
    c0 = (wid % CHUNKS_PER_ROW) * TOK_PER_W

    pltpu.sync_copy(ids_hbm.at[b, pl.ds(c0, TOK_PER_W)], idxc_v)
    ids_cp = pltpu.async_copy(ids_hbm.at[b], ids_v, sem_p)

    def gather(sc, buf):
        idx_ref = idxc_v.at[pl.ds(sc * SUB, SUB)]
        return pltpu.async_copy(table_hbm.at[idx_ref], rows_v.at[buf], sem_g)

    def store(sc, buf):
        dst = out_hbm.at[pl.ds(b * S + c0 + sc * SUB, SUB)]
        return pltpu.async_copy(rows_v.at[buf], dst, sem_s)

    gd = [gather(sc, sc % NBUF) for sc in range(PRIME)]
    ids_cp.wait()

    lane = lax.iota(jnp.int32, 16)

    def count_body(t, base_vec):
        m = ids_v[pl.ds(t * 16, 16)] == PID
        return base_vec + plsc.all_reduce_population_count(m)

    base_vec = lax.fori_loop(0, c0 // 16, count_body,
                             jnp.zeros((16,), jnp.int32))

    def pos_body(t, cnt_vec):
        m = idxc_v[pl.ds(t * 16, 16)] == PID
        col = c0 + t * 16 + lane
        pref = plsc.cumsum(jnp.where(m, 1, 0))
        slot = jnp.where(m, cnt_vec + pref - 1, 0)
        plsc.store_scatter(pos_v, [slot], col, mask=m)
        return cnt_vec + plsc.all_reduce_population_count(m)

    cnt_vec = lax.fori_loop(0, TOK_PER_W // 16, pos_body,
                            jnp.zeros((16,), jnp.int32))
    base = base_vec[0]
    cnt = cnt_vec[0]

    valid0 = lane < cnt_vec
    rank0 = jnp.where(valid0, base_vec + lane, 0)
    pg = pltpu.async_copy(prompt_hbm.at[rank0], pv_v, sem_p)

    sd = [None] * N_SUB
    waited = set()
    for sc in range(N_SUB):
        gd[sc].wait()
        sd[sc] = store(sc, sc % NBUF)
        nx = sc + PRIME
        if nx < N_SUB:
            if nx - NBUF >= 0:
                sd[nx - NBUF].wait()
                waited.add(nx - NBUF)
            gd.append(gather(nx, nx % NBUF))
    for sc in range(N_SUB):
        if sc not in waited:
            sd[sc].wait()

    pg.wait()

    @pl.when(cnt > 0)
    def _():
        pos16 = pos_v[pl.ds(0, 16)]
        pos_sel = jnp.where(valid0, pos16, pos16[0])
        src_sel = jnp.where(valid0, lane, 0)
        ow = []
        for k in range(16):
            dst = out_hbm.at[pl.ds(b * S + pos_sel[k], 1)]
            desc = pltpu.make_async_copy(
                pv_v.at[pl.ds(src_sel[k], 1)], dst, sem_p)
            ow.append(desc)
            desc.start()
        for k in range(16):
            ow[k].wait()

    def slow_body(k, carry):
        rank = base + k
        posk = plsc.load_gather(pos_v, [jnp.zeros((16,), jnp.int32) + k])[0]
        pltpu.sync_copy(prompt_hbm.at[pl.ds(rank, 1)], pv_v.at[pl.ds(0, 1)])
        pltpu.sync_copy(pv_v.at[pl.ds(0, 1)],
                        out_hbm.at[pl.ds(b * S + posk, 1)])
        return carry

    lax.fori_loop(16, jnp.maximum(cnt, 16), slow_body, jnp.int32(0))


@functools.partial(
    pl.kernel,
    mesh=plsc.VectorSubcoreMesh(core_axis_name="c", subcore_axis_name="s"),
    compiler_params=pltpu.CompilerParams(needs_layout_passes=False),
    out_type=jax.ShapeDtypeStruct((B * S, D), jnp.float32),
    scratch_types=[
        pltpu.VMEM((S,), jnp.int32),
        pltpu.VMEM((TOK_PER_W,), jnp.int32),
        pltpu.VMEM((NBUF, SUB, D), jnp.float32),
        pltpu.VMEM((MAX_P,), jnp.int32),
        pltpu.VMEM((16, D), jnp.float32),
        pltpu.SemaphoreType.DMA,
        pltpu.SemaphoreType.DMA,
        pltpu.SemaphoreType.DMA,
    ],
)
def _sc_embed(ids_hbm, table_hbm, prompt_hbm, out_hbm,
              ids_v, idxc_v, rows_v, pos_v, pv_v, sem_g, sem_s, sem_p):
    _worker_body(ids_hbm, table_hbm, prompt_hbm, out_hbm,
                 ids_v, idxc_v, rows_v, pos_v, pv_v, sem_g, sem_s, sem_p)


def kernel(input_ids, bert_embedding_weight, prompt):
    out = _sc_embed(input_ids, bert_embedding_weight, prompt)
    return out.reshape(B, S, D)


# speedup vs baseline: 1.1658x; 1.1658x over previous
import functools

import jax
import jax.numpy as jnp
from jax import lax
from jax.experimental import pallas as pl
from jax.experimental.pallas import tpu as pltpu
from jax.experimental.pallas import tpu_sc as plsc

B, S, D = 4, 2048, 768
VOCAB = 100000
PROMPT_LEN = 50
PID = 1

NW = 32
TOK_PER_W = (B * S) // NW
CHUNKS_PER_ROW = S // TOK_PER_W
SUB = 16
N_SUB = TOK_PER_W // SUB
NBUF = 8
PRIME = 4
MAX_P = 64
N_GRP = MAX_P // 16


def _worker_body(ids_hbm, table_hbm, prompt_hbm, out_hbm,
                 ids_v, idxc_v, rows_v, pos_v, pv_v, sem_g, sem_s, sem_p):
    wid = lax.axis_index("s") * 2 + lax.axis_index("c")
    b = wid // CHUNKS_PER_ROW
    c0 = (wid % CHUNKS_PER_ROW) * TOK_PER_W

    pltpu.sync_copy(ids_hbm.at[b, pl.ds(c0, TOK_PER_W)], idxc_v)
    ids_cp = pltpu.async_copy(ids_hbm.at[b], ids_v, sem_p)

    def gather(sc, buf):
        idx_ref = idxc_v.at[pl.ds(sc * SUB, SUB)]
        return pltpu.async_copy(table_hbm.at[idx_ref], rows_v.at[buf], sem_g)

    def store(sc, buf):
        dst = out_hbm.at[pl.ds(b * S + c0 + sc * SUB, SUB)]
        return pltpu.async_copy(rows_v.at[buf], dst, sem_s)

    gd = [gather(sc, sc % NBUF) for sc in range(PRIME)]
    ids_cp.wait()

    lane = lax.iota(jnp.int32, 16)

    def count_body(t, base_vec):
        for u in range(4):
            m = ids_v[pl.ds(t * 64 + u * 16, 16)] == PID
            base_vec = base_vec + plsc.all_reduce_population_count(m)
        return base_vec

    base_vec = lax.fori_loop(0, c0 // 64, count_body,
                             jnp.zeros((16,), jnp.int32))

    def pos_body(t, cnt_vec):
        m = idxc_v[pl.ds(t * 16, 16)] == PID
        col = c0 + t * 16 + lane
        pref = plsc.cumsum(jnp.where(m, 1, 0))
        slot = jnp.where(m, cnt_vec + pref - 1, 0)
        plsc.store_scatter(pos_v, [slot], col, mask=m)
        return cnt_vec + plsc.all_reduce_population_count(m)

    cnt_vec = lax.fori_loop(0, TOK_PER_W // 16, pos_body,
                            jnp.zeros((16,), jnp.int32))
    base = base_vec[0]
    cnt = cnt_vec[0]

    valid0 = lane < cnt_vec
    rank0 = jnp.where(valid0, base_vec + lane, base_vec)
    pg = pltpu.async_copy(prompt_hbm.at[rank0], pv_v, sem_p)

    sd = [None] * N_SUB
    waited = set()
    for sc in range(N_SUB):
        gd[sc].wait()
        sd[sc] = store(sc, sc % NBUF)
        nx = sc + PRIME
        if nx < N_SUB:
            if nx - NBUF >= 0:
                sd[nx - NBUF].wait()
                waited.add(nx - NBUF)
            gd.append(gather(nx, nx % NBUF))
    for sc in range(N_SUB):
        if sc not in waited:
            sd[sc].wait()

    pg.wait()

    @pl.when(cnt > 0)
    def _():
        pos16 = pos_v[pl.ds(0, 16)]
        dest0 = b * S + jnp.where(valid0, pos16, pos16[0])
        pltpu.async_copy(pv_v, out_hbm.at[dest0], sem_p).wait()

    def slow_body(k, carry):
        rank = base + k
        posk = plsc.load_gather(pos_v, [jnp.zeros((16,), jnp.int32) + k])[0]
        pltpu.sync_copy(prompt_hbm.at[pl.ds(rank, 1)], pv_v.at[pl.ds(0, 1)])
        pltpu.sync_copy(pv_v.at[pl.ds(0, 1)],
                        out_hbm.at[pl.ds(b * S + posk, 1)])
        return carry

    lax.fori_loop(16, jnp.maximum(cnt, 16), slow_body, jnp.int32(0))


@functools.partial(
    pl.kernel,
    mesh=plsc.VectorSubcoreMesh(core_axis_name="c", subcore_axis_name="s"),
    compiler_params=pltpu.CompilerParams(needs_layout_passes=False),
    out_type=jax.ShapeDtypeStruct((B * S, D), jnp.float32),
    scratch_types=[
        pltpu.VMEM((S,), jnp.int32),
        pltpu.VMEM((TOK_PER_W,), jnp.int32),
        pltpu.VMEM((NBUF, SUB, D), jnp.float32),
        pltpu.VMEM((MAX_P,), jnp.int32),
        pltpu.VMEM((16, D), jnp.float32),
        pltpu.SemaphoreType.DMA,
        pltpu.SemaphoreType.DMA,
        pltpu.SemaphoreType.DMA,
    ],
)
def _sc_embed(ids_hbm, table_hbm, prompt_hbm, out_hbm,
              ids_v, idxc_v, rows_v, pos_v, pv_v, sem_g, sem_s, sem_p):
    _worker_body(ids_hbm, table_hbm, prompt_hbm, out_hbm,
                 ids_v, idxc_v, rows_v, pos_v, pv_v, sem_g, sem_s, sem_p)


def kernel(input_ids, bert_embedding_weight, prompt):
    out = _sc_embed(input_ids, bert_embedding_weight, prompt)
    return out.reshape(B, S, D)


# speedup vs baseline: 1.1675x; 1.0015x over previous
import functools

import jax
import jax.numpy as jnp
from jax import lax
from jax.experimental import pallas as pl
from jax.experimental.pallas import tpu as pltpu
from jax.experimental.pallas import tpu_sc as plsc

B, S, D = 4, 2048, 768
VOCAB = 100000
PROMPT_LEN = 50
PID = 1

NW = 32
TOK_PER_W = (B * S) // NW
CHUNKS_PER_ROW = S // TOK_PER_W
SUB = 16
N_SUB = TOK_PER_W // SUB
NBUF = 8
PRIME = 6
MAX_P = 64
N_GRP = MAX_P // 16


def _worker_body(ids_hbm, table_hbm, prompt_hbm, out_hbm,
                 ids_v, idxc_v, rows_v, pos_v, pv_v, sem_g, sem_s, sem_p):
    wid = lax.axis_index("s") * 2 + lax.axis_index("c")
    b = wid // CHUNKS_PER_ROW
    c0 = (wid % CHUNKS_PER_ROW) * TOK_PER_W

    pltpu.sync_copy(ids_hbm.at[b, pl.ds(c0, TOK_PER_W)], idxc_v)
    ids_cp = pltpu.async_copy(ids_hbm.at[b], ids_v, sem_p)

    def gather(sc, buf):
        idx_ref = idxc_v.at[pl.ds(sc * SUB, SUB)]
        return pltpu.async_copy(table_hbm.at[idx_ref], rows_v.at[buf], sem_g)

    def store(sc, buf):
        dst = out_hbm.at[pl.ds(b * S + c0 + sc * SUB, SUB)]
        return pltpu.async_copy(rows_v.at[buf], dst, sem_s)

    gd = [gather(sc, sc % NBUF) for sc in range(PRIME)]
    ids_cp.wait()

    lane = lax.iota(jnp.int32, 16)

    def count_body(t, base_vec):
        for u in range(4):
            m = ids_v[pl.ds(t * 64 + u * 16, 16)] == PID
            base_vec = base_vec + plsc.all_reduce_population_count(m)
        return base_vec

    base_vec = lax.fori_loop(0, c0 // 64, count_body,
                             jnp.zeros((16,), jnp.int32))

    def pos_body(t, cnt_vec):
        m = idxc_v[pl.ds(t * 16, 16)] == PID
        col = c0 + t * 16 + lane
        pref = plsc.cumsum(jnp.where(m, 1, 0))
        slot = jnp.where(m, cnt_vec + pref - 1, 0)
        plsc.store_scatter(pos_v, [slot], col, mask=m)
        return cnt_vec + plsc.all_reduce_population_count(m)

    cnt_vec = lax.fori_loop(0, TOK_PER_W // 16, pos_body,
                            jnp.zeros((16,), jnp.int32))
    base = base_vec[0]
    cnt = cnt_vec[0]

    valid0 = lane < cnt_vec
    rank0 = jnp.where(valid0, base_vec + lane, base_vec)
    pg = pltpu.async_copy(prompt_hbm.at[rank0], pv_v, sem_p)

    sd = [None] * N_SUB
    waited = set()
    for sc in range(N_SUB):
        gd[sc].wait()
        sd[sc] = store(sc, sc % NBUF)
        nx = sc + PRIME
        if nx < N_SUB:
            if nx - NBUF >= 0:
                sd[nx - NBUF].wait()
                waited.add(nx - NBUF)
            gd.append(gather(nx, nx % NBUF))
    for sc in range(N_SUB):
        if sc not in waited:
            sd[sc].wait()

    pg.wait()

    @pl.when(cnt > 0)
    def _():
        pos16 = pos_v[pl.ds(0, 16)]
        dest0 = b * S + jnp.where(valid0, pos16, pos16[0])
        pltpu.async_copy(pv_v, out_hbm.at[dest0], sem_p).wait()

    def slow_body(k, carry):
        rank = base + k
        posk = plsc.load_gather(pos_v, [jnp.zeros((16,), jnp.int32) + k])[0]
        pltpu.sync_copy(prompt_hbm.at[pl.ds(rank, 1)], pv_v.at[pl.ds(0, 1)])
        pltpu.sync_copy(pv_v.at[pl.ds(0, 1)],
                        out_hbm.at[pl.ds(b * S + posk, 1)])
        return carry

    lax.fori_loop(16, jnp.maximum(cnt, 16), slow_body, jnp.int32(0))


@functools.partial(
    pl.kernel,
    mesh=plsc.VectorSubcoreMesh(core_axis_name="c", subcore_axis_name="s"),
    compiler_params=pltpu.CompilerParams(needs_layout_passes=False),
    out_type=jax.ShapeDtypeStruct((B * S, D), jnp.float32),
    scratch_types=[
        pltpu.VMEM((S,), jnp.int32),
        pltpu.VMEM((TOK_PER_W,), jnp.int32),
        pltpu.VMEM((NBUF, SUB, D), jnp.float32),
        pltpu.VMEM((MAX_P,), jnp.int32),
        pltpu.VMEM((16, D), jnp.float32),
        pltpu.SemaphoreType.DMA,
        pltpu.SemaphoreType.DMA,
        pltpu.SemaphoreType.DMA,
    ],
)
def _sc_embed(ids_hbm, table_hbm, prompt_hbm, out_hbm,
              ids_v, idxc_v, rows_v, pos_v, pv_v, sem_g, sem_s, sem_p):
    _worker_body(ids_hbm, table_hbm, prompt_hbm, out_hbm,
                 ids_v, idxc_v, rows_v, pos_v, pv_v, sem_g, sem_s, sem_p)


def kernel(input_ids, bert_embedding_weight, prompt):
    out = _sc_embed(input_ids, bert_embedding_weight, prompt)
    return out.reshape(B, S, D)


# speedup vs baseline: 1.1697x; 1.0019x over previous
import functools

import jax
import jax.numpy as jnp
from jax import lax
from jax.experimental import pallas as pl
from jax.experimental.pallas import tpu as pltpu
from jax.experimental.pallas import tpu_sc as plsc

B, S, D = 4, 2048, 768
VOCAB = 100000
PROMPT_LEN = 50
PID = 1

NW = 32
TOK_PER_W = (B * S) // NW
CHUNKS_PER_ROW = S // TOK_PER_W
SUB = 16
N_SUB = TOK_PER_W // SUB
NBUF = 8
PRIME = 6
MAX_P = 64
N_GRP = MAX_P // 16


def _worker_body(ids_hbm, table_hbm, prompt_hbm, out_hbm,
                 ids_v, idxc_v, rows_v, pos_v, pv_v, sem_g, sem_s, sem_p):
    wid = lax.axis_index("s") * 2 + lax.axis_index("c")
    b = wid // CHUNKS_PER_ROW
    c0 = (wid % CHUNKS_PER_ROW) * TOK_PER_W

    pltpu.sync_copy(ids_hbm.at[b, pl.ds(c0, TOK_PER_W)], idxc_v)
    ids_cp = pltpu.async_copy(ids_hbm.at[b], ids_v, sem_p)

    def gather(sc, buf):
        idx_ref = idxc_v.at[pl.ds(sc * SUB, SUB)]
        return pltpu.async_copy(table_hbm.at[idx_ref], rows_v.at[buf],
                                sem_g.at[buf])

    def store(sc, buf):
        dst = out_hbm.at[pl.ds(b * S + c0 + sc * SUB, SUB)]
        return pltpu.async_copy(rows_v.at[buf], dst, sem_s.at[buf])

    gd = [gather(sc, sc % NBUF) for sc in range(PRIME)]
    ids_cp.wait()

    lane = lax.iota(jnp.int32, 16)

    def count_body(t, base_vec):
        for u in range(4):
            m = ids_v[pl.ds(t * 64 + u * 16, 16)] == PID
            base_vec = base_vec + plsc.all_reduce_population_count(m)
        return base_vec

    base_vec = lax.fori_loop(0, c0 // 64, count_body,
                             jnp.zeros((16,), jnp.int32))

    def pos_body(t, cnt_vec):
        m = idxc_v[pl.ds(t * 16, 16)] == PID
        col = c0 + t * 16 + lane
        pref = plsc.cumsum(jnp.where(m, 1, 0))
        slot = jnp.where(m, cnt_vec + pref - 1, 0)
        plsc.store_scatter(pos_v, [slot], col, mask=m)
        return cnt_vec + plsc.all_reduce_population_count(m)

    cnt_vec = lax.fori_loop(0, TOK_PER_W // 16, pos_body,
                            jnp.zeros((16,), jnp.int32))
    base = base_vec[0]
    cnt = cnt_vec[0]

    valid0 = lane < cnt_vec
    rank0 = jnp.where(valid0, base_vec + lane, base_vec)
    pg = pltpu.async_copy(prompt_hbm.at[rank0], pv_v, sem_p)

    sd = [None] * N_SUB
    waited = set()
    for sc in range(N_SUB):
        gd[sc].wait()
        sd[sc] = store(sc, sc % NBUF)
        nx = sc + PRIME
        if nx < N_SUB:
            if nx - NBUF >= 0:
                sd[nx - NBUF].wait()
                waited.add(nx - NBUF)
            gd.append(gather(nx, nx % NBUF))
    for sc in range(N_SUB):
        if sc not in waited:
            sd[sc].wait()

    pg.wait()

    @pl.when(cnt > 0)
    def _():
        pos16 = pos_v[pl.ds(0, 16)]
        dest0 = b * S + jnp.where(valid0, pos16, pos16[0])
        pltpu.async_copy(pv_v, out_hbm.at[dest0], sem_p).wait()

    def slow_body(k, carry):
        rank = base + k
        posk = plsc.load_gather(pos_v, [jnp.zeros((16,), jnp.int32) + k])[0]
        pltpu.sync_copy(prompt_hbm.at[pl.ds(rank, 1)], pv_v.at[pl.ds(0, 1)])
        pltpu.sync_copy(pv_v.at[pl.ds(0, 1)],
                        out_hbm.at[pl.ds(b * S + posk, 1)])
        return carry

    lax.fori_loop(16, jnp.maximum(cnt, 16), slow_body, jnp.int32(0))


@functools.partial(
    pl.kernel,
    mesh=plsc.VectorSubcoreMesh(core_axis_name="c", subcore_axis_name="s"),
    compiler_params=pltpu.CompilerParams(needs_layout_passes=False),
    out_type=jax.ShapeDtypeStruct((B * S, D), jnp.float32),
    scratch_types=[
        pltpu.VMEM((S,), jnp.int32),
        pltpu.VMEM((TOK_PER_W,), jnp.int32),
        pltpu.VMEM((NBUF, SUB, D), jnp.float32),
        pltpu.VMEM((MAX_P,), jnp.int32),
        pltpu.VMEM((16, D), jnp.float32),
        pltpu.SemaphoreType.DMA((NBUF,)),
        pltpu.SemaphoreType.DMA((NBUF,)),
        pltpu.SemaphoreType.DMA,
    ],
)
def _sc_embed(ids_hbm, table_hbm, prompt_hbm, out_hbm,
              ids_v, idxc_v, rows_v, pos_v, pv_v, sem_g, sem_s, sem_p):
    _worker_body(ids_hbm, table_hbm, prompt_hbm, out_hbm,
                 ids_v, idxc_v, rows_v, pos_v, pv_v, sem_g, sem_s, sem_p)


def kernel(input_ids, bert_embedding_weight, prompt):
    out = _sc_embed(input_ids, bert_embedding_weight, prompt)
    return out.reshape(B, S, D)


# speedup vs baseline: 1.1866x; 1.0145x over previous
import functools

import jax
import jax.numpy as jnp
from jax import lax
from jax.experimental import pallas as pl
from jax.experimental.pallas import tpu as pltpu
from jax.experimental.pallas import tpu_sc as plsc

B, S, D = 4, 2048, 768
VOCAB = 100000
PROMPT_LEN = 50
PID = 1

NW = 32
TOK_PER_W = (B * S) // NW
CHUNKS_PER_ROW = S // TOK_PER_W
SUB = 32
N_SUB = TOK_PER_W // SUB
NBUF = 4
PRIME = 3
MAX_P = 64
N_GRP = MAX_P // 16


def _worker_body(ids_hbm, table_hbm, prompt_hbm, out_hbm,
                 ids_v, idxc_v, rows_v, pos_v, pv_v, sem_g, sem_s, sem_p):
    wid = lax.axis_index("s") * 2 + lax.axis_index("c")
    b = wid // CHUNKS_PER_ROW
    c0 = (wid % CHUNKS_PER_ROW) * TOK_PER_W

    pltpu.sync_copy(ids_hbm.at[b, pl.ds(c0, TOK_PER_W)], idxc_v)
    ids_cp = pltpu.async_copy(ids_hbm.at[b], ids_v, sem_p)

    def gather(sc, buf):
        idx_ref = idxc_v.at[pl.ds(sc * SUB, SUB)]
        return pltpu.async_copy(table_hbm.at[idx_ref], rows_v.at[buf],
                                sem_g.at[buf])

    def store(sc, buf):
        dst = out_hbm.at[pl.ds(b * S + c0 + sc * SUB, SUB)]
        return pltpu.async_copy(rows_v.at[buf], dst, sem_s.at[buf])

    gd = [gather(sc, sc % NBUF) for sc in range(PRIME)]
    ids_cp.wait()

    lane = lax.iota(jnp.int32, 16)

    def count_body(t, base_vec):
        for u in range(4):
            m = ids_v[pl.ds(t * 64 + u * 16, 16)] == PID
            base_vec = base_vec + plsc.all_reduce_population_count(m)
        return base_vec

    base_vec = lax.fori_loop(0, c0 // 64, count_body,
                             jnp.zeros((16,), jnp.int32))

    def pos_body(t, cnt_vec):
        m = idxc_v[pl.ds(t * 16, 16)] == PID
        col = c0 + t * 16 + lane
        pref = plsc.cumsum(jnp.where(m, 1, 0))
        slot = jnp.where(m, cnt_vec + pref - 1, 0)
        plsc.store_scatter(pos_v, [slot], col, mask=m)
        return cnt_vec + plsc.all_reduce_population_count(m)

    cnt_vec = lax.fori_loop(0, TOK_PER_W // 16, pos_body,
                            jnp.zeros((16,), jnp.int32))
    base = base_vec[0]
    cnt = cnt_vec[0]

    valid0 = lane < cnt_vec
    rank0 = jnp.where(valid0, base_vec + lane, base_vec)
    pg = pltpu.async_copy(prompt_hbm.at[rank0], pv_v, sem_p)

    sd = [None] * N_SUB
    waited = set()
    for sc in range(N_SUB):
        gd[sc].wait()
        sd[sc] = store(sc, sc % NBUF)
        nx = sc + PRIME
        if nx < N_SUB:
            if nx - NBUF >= 0:
                sd[nx - NBUF].wait()
                waited.add(nx - NBUF)
            gd.append(gather(nx, nx % NBUF))
    for sc in range(N_SUB):
        if sc not in waited:
            sd[sc].wait()

    pg.wait()

    @pl.when(cnt > 0)
    def _():
        pos16 = pos_v[pl.ds(0, 16)]
        dest0 = b * S + jnp.where(valid0, pos16, pos16[0])
        pltpu.async_copy(pv_v, out_hbm.at[dest0], sem_p).wait()

    def slow_body(k, carry):
        rank = base + k
        posk = plsc.load_gather(pos_v, [jnp.zeros((16,), jnp.int32) + k])[0]
        pltpu.sync_copy(prompt_hbm.at[pl.ds(rank, 1)], pv_v.at[pl.ds(0, 1)])
        pltpu.sync_copy(pv_v.at[pl.ds(0, 1)],
                        out_hbm.at[pl.ds(b * S + posk, 1)])
        return carry

    lax.fori_loop(16, jnp.maximum(cnt, 16), slow_body, jnp.int32(0))


@functools.partial(
    pl.kernel,
    mesh=plsc.VectorSubcoreMesh(core_axis_name="c", subcore_axis_name="s"),
    compiler_params=pltpu.CompilerParams(needs_layout_passes=False),
    out_type=jax.ShapeDtypeStruct((B * S, D), jnp.float32),
    scratch_types=[
        pltpu.VMEM((S,), jnp.int32),
        pltpu.VMEM((TOK_PER_W,), jnp.int32),
        pltpu.VMEM((NBUF, SUB, D), jnp.float32),
        pltpu.VMEM((MAX_P,), jnp.int32),
        pltpu.VMEM((16, D), jnp.float32),
        pltpu.SemaphoreType.DMA((NBUF,)),
        pltpu.SemaphoreType.DMA((NBUF,)),
        pltpu.SemaphoreType.DMA,
    ],
)
def _sc_embed(ids_hbm, table_hbm, prompt_hbm, out_hbm,
              ids_v, idxc_v, rows_v, pos_v, pv_v, sem_g, sem_s, sem_p):
    _worker_body(ids_hbm, table_hbm, prompt_hbm, out_hbm,
                 ids_v, idxc_v, rows_v, pos_v, pv_v, sem_g, sem_s, sem_p)


def kernel(input_ids, bert_embedding_weight, prompt):
    out = _sc_embed(input_ids, bert_embedding_weight, prompt)
    return out.reshape(B, S, D)
